# Initial kernel scaffold; baseline (speedup 1.0000x reference)
import jax
import jax.numpy as jnp
from jax.experimental import pallas as pl


def kernel(patches):
    raise NotImplementedError("write your pallas kernel here")


# speedup vs baseline: 7.8850x; 7.8850x over previous
import jax
import jax.numpy as jnp
from jax.experimental import pallas as pl

_T = 256
_B = 1024
_C = 192
_REMAIN = 64
_BT = 8


def _shuffle_kernel(p_ref, out_ref, idx_ref):
    i = pl.program_id(0)
    for k in range(_BT):
        out_ref[k, :, :] = p_ref[_BT - 1 - k, :, :]
    rows_per_step = _T // (_REMAIN // _BT)
    row = i * rows_per_step + jax.lax.broadcasted_iota(
        jnp.int32, (rows_per_step, _B), 0)
    idx_ref[...] = (_T - 1) - row


def kernel(patches):
    n_steps = _REMAIN // _BT
    idx_rows = _T // n_steps
    out, idx = pl.pallas_call(
        _shuffle_kernel,
        grid=(n_steps,),
        in_specs=[
            pl.BlockSpec((_BT, _B, _C), lambda i: (n_steps * 4 - 1 - i, 0, 0)),
        ],
        out_specs=[
            pl.BlockSpec((_BT, _B, _C), lambda i: (i, 0, 0)),
            pl.BlockSpec((idx_rows, _B), lambda i: (i, 0)),
        ],
        out_shape=[
            jax.ShapeDtypeStruct((_REMAIN, _B, _C), patches.dtype),
            jax.ShapeDtypeStruct((_T, _B), jnp.int32),
        ],
    )(patches)
    return (out, idx, idx)
